# baseline (speedup 1.0000x reference)
import functools

import jax
import jax.numpy as jnp
from jax.experimental import pallas as pl

_B, _N = 4096, 50
_D1, _D2 = 32, 32
_H, _O, _R = 64, 32, 8


def _leaky(x):
    return jnp.where(x > 0, x, 0.01 * x)


def _fused_kernel(u_ref, i_ref, a_ref, o_ref, s_ref,
                  W1c_ref, b1c_ref, W2c_ref, b2c_ref,
                  Wu1_ref, bu1_ref, Wu2_ref, bu2_ref,
                  out_ref):
    bb, n, d1 = a_ref.shape
    rows = bb * n
    a = a_ref[...].reshape(rows, d1)
    o = o_ref[...].reshape(rows, d1)
    ao = jnp.concatenate([a, o], axis=-1)
    s = s_ref[...]

    h_all = _leaky(jnp.dot(ao, W1c_ref[...],
                           preferred_element_type=jnp.float32) + b1c_ref[...])
    h_sel = jnp.zeros((rows, _H), dtype=jnp.float32)
    for r in range(_R):
        h_sel += jnp.where(s == r, h_all[:, r * _H:(r + 1) * _H], 0.0)

    z_all = _leaky(jnp.dot(h_sel, W2c_ref[...],
                           preferred_element_type=jnp.float32) + b2c_ref[...])
    o_sel = jnp.zeros((rows, _O), dtype=jnp.float32)
    for r in range(_R):
        o_sel += jnp.where(s == r, z_all[:, r * _O:(r + 1) * _O], 0.0)

    ui_in = jnp.concatenate([u_ref[...], i_ref[...]], axis=-1)
    hu = _leaky(jnp.dot(ui_in, Wu1_ref[...],
                        preferred_element_type=jnp.float32) + bu1_ref[...])
    ue = _leaky(jnp.dot(hu, Wu2_ref[...],
                        preferred_element_type=jnp.float32) + bu2_ref[...])

    ue_rows = jnp.broadcast_to(ue[:, None, :], (bb, n, _O)).reshape(rows, _O)
    out_ref[...] = jnp.sum(o_sel * ue_rows, axis=-1, keepdims=True)


@jax.jit
def kernel(u_emb, i_emb, a_emb, o_emb, s,
           W_ui1, b_ui1, W_ui2, b_ui2, W_ao1, b_ao1, W_ao2, b_ao2):
    BB = 64
    grid = (_B // BB,)
    s_flat = s.reshape(_B * _N, 1)

    W1c = jnp.transpose(W_ao1, (1, 0, 2)).reshape(2 * _D1, _R * _H)
    b1c = b_ao1.reshape(1, _R * _H)
    W2c = jnp.transpose(W_ao2, (1, 0, 2)).reshape(_H, _R * _O)
    b2c = b_ao2.reshape(1, _R * _O)
    bu1 = b_ui1.reshape(1, _H)
    bu2 = b_ui2.reshape(1, _O)

    full = lambda *shape: pl.BlockSpec(shape, lambda i: (0,) * len(shape))
    out = pl.pallas_call(
        _fused_kernel,
        grid=grid,
        in_specs=[
            pl.BlockSpec((BB, _D2), lambda i: (i, 0)),
            pl.BlockSpec((BB, _D2), lambda i: (i, 0)),
            pl.BlockSpec((BB, _N, _D1), lambda i: (i, 0, 0)),
            pl.BlockSpec((BB, _N, _D1), lambda i: (i, 0, 0)),
            pl.BlockSpec((BB * _N, 1), lambda i: (i, 0)),
            full(2 * _D1, _R * _H),
            full(1, _R * _H),
            full(_H, _R * _O),
            full(1, _R * _O),
            full(2 * _D2, _H),
            full(1, _H),
            full(_H, _O),
            full(1, _O),
        ],
        out_specs=pl.BlockSpec((BB * _N, 1), lambda i: (i, 0)),
        out_shape=jax.ShapeDtypeStruct((_B * _N, 1), jnp.float32),
    )(u_emb, i_emb, a_emb, o_emb, s_flat, W1c, b1c, W2c, b2c,
      W_ui1, bu1, W_ui2, bu2)
    return out.reshape(_B, _N)
